# Initial kernel scaffold; baseline (speedup 1.0000x reference)
import jax
import jax.numpy as jnp
from jax.experimental import pallas as pl


def kernel(boxes, scores):
    raise NotImplementedError("write your pallas kernel here")


# speedup vs baseline: 13.4481x; 13.4481x over previous
import jax
import jax.numpy as jnp
from jax.experimental import pallas as pl
from jax.experimental.pallas import tpu as pltpu

_PRE = 2000
_S = 128
_B = 16
_PAD = _B * _S
_OUT = 300
_T = 0.5


def _nms_kernel(x1r, y1r, x2r, y2r, x1c, y1c, x2c, y2c, sup_ref, iou_scr):
    iota = jax.lax.broadcasted_iota(jnp.int32, (1, _S), 1)
    for b in range(_B):
        bx1r = x1r[pl.ds(b, 1), :]
        by1r = y1r[pl.ds(b, 1), :]
        bx2r = x2r[pl.ds(b, 1), :]
        by2r = y2r[pl.ds(b, 1), :]
        area_br = (bx2r - bx1r) * (by2r - by1r)

        bx1c = x1c[pl.ds(b * _S, _S), :]
        by1c = y1c[pl.ds(b * _S, _S), :]
        bx2c = x2c[pl.ds(b * _S, _S), :]
        by2c = y2c[pl.ds(b * _S, _S), :]
        area_bc = (bx2c - bx1c) * (by2c - by1c)

        w = jnp.maximum(jnp.minimum(bx2c, bx2r) - jnp.maximum(bx1c, bx1r), 0.0)
        h = jnp.maximum(jnp.minimum(by2c, by2r) - jnp.maximum(by1c, by1r), 0.0)
        inter = w * h
        union = area_bc + area_br - inter
        iou = inter / jnp.maximum(union, 1e-9)
        iou_scr[...] = jnp.where(iou > _T, 1.0, 0.0)

        def cross_body(a, acc):
            ax1c = x1c[pl.ds(a * _S, _S), :]
            ay1c = y1c[pl.ds(a * _S, _S), :]
            ax2c = x2c[pl.ds(a * _S, _S), :]
            ay2c = y2c[pl.ds(a * _S, _S), :]
            area_ac = (ax2c - ax1c) * (ay2c - ay1c)
            w = jnp.maximum(jnp.minimum(ax2c, bx2r) - jnp.maximum(ax1c, bx1r), 0.0)
            h = jnp.maximum(jnp.minimum(ay2c, by2r) - jnp.maximum(ay1c, by1r), 0.0)
            inter = w * h
            union = area_ac + area_br - inter
            m = jnp.where(inter / jnp.maximum(union, 1e-9) > _T, 1.0, 0.0)
            keep_a = 1.0 - sup_ref[pl.ds(a, 1), :]
            return acc + jax.lax.dot_general(
                keep_a, m, (((1,), (0,)), ((), ())),
                preferred_element_type=jnp.float32)

        if b > 0:
            acc = jax.lax.fori_loop(0, b, cross_body,
                                    jnp.zeros((1, _S), jnp.float32))
            sup0 = jnp.where(acc > 0.0, 1.0, 0.0)
        else:
            sup0 = jnp.zeros((1, _S), jnp.float32)

        def within_body(i, sup):
            row = iou_scr[pl.ds(i, 1), :]
            si = jnp.sum(sup * jnp.where(iota == i, 1.0, 0.0))
            later = jnp.where(iota > i, 1.0, 0.0)
            add = row * later * jnp.where(si < 0.5, 1.0, 0.0)
            return jnp.maximum(sup, add)

        sup_b = jax.lax.fori_loop(0, _S, within_body, sup0)
        sup_ref[pl.ds(b, 1), :] = sup_b


def kernel(boxes, scores):
    top_scores, top_idx = jax.lax.top_k(scores, _PRE)
    top_boxes = jnp.take(boxes, top_idx, axis=0)
    pb = jnp.pad(top_boxes, ((0, _PAD - _PRE), (0, 0)))
    x1, y1, x2, y2 = pb[:, 0], pb[:, 1], pb[:, 2], pb[:, 3]

    def row(v):
        return v.reshape(_B, _S)

    def col(v):
        return v.reshape(_PAD, 1)

    sup = pl.pallas_call(
        _nms_kernel,
        out_shape=jax.ShapeDtypeStruct((_B, _S), jnp.float32),
        scratch_shapes=[pltpu.VMEM((_S, _S), jnp.float32)],
    )(row(x1), row(y1), row(x2), row(y2),
      col(x1), col(y1), col(x2), col(y2))

    keep = sup.reshape(_PAD)[:_PRE] < 0.5
    masked = jnp.where(keep, top_scores, jnp.full_like(top_scores, -1e6))
    out_s, out_i = jax.lax.top_k(masked, _OUT)
    out_b = jnp.take(top_boxes, out_i, axis=0)
    return jnp.concatenate([out_s[:, None], out_b], axis=1)
